# baseline (speedup 1.0000x reference)
import functools

import jax
import jax.numpy as jnp
from jax.experimental import pallas as pl

N, E, D, L = 10000, 160000, 128, 4


def _mm_kernel(x_ref, w_ref, b_ref, o_ref):
    o_ref[...] = (
        jnp.dot(x_ref[...], w_ref[...], preferred_element_type=jnp.float32)
        + b_ref[...]
    )


def _mm(x, w, b, blk):
    m = x.shape[0]
    return pl.pallas_call(
        _mm_kernel,
        grid=(m // blk,),
        in_specs=[
            pl.BlockSpec((blk, D), lambda i: (i, 0)),
            pl.BlockSpec((D, D), lambda i: (0, 0)),
            pl.BlockSpec((1, D), lambda i: (0, 0)),
        ],
        out_specs=pl.BlockSpec((blk, D), lambda i: (i, 0)),
        out_shape=jax.ShapeDtypeStruct((m, D), jnp.float32),
    )(x, w, b.reshape(1, D))


def _bn(x, gamma, beta, eps=1e-5):
    mu = jnp.mean(x, axis=0, keepdims=True)
    var = jnp.var(x, axis=0, keepdims=True)
    return (x - mu) / jnp.sqrt(var + eps) * gamma + beta


def kernel(h, e, edge_index, A1_w, A1_b, A2_w, A2_b, A3_w, A3_b, B1_w, B1_b,
           B2_w, B2_b, B3_w, B3_b, bn_h_g, bn_h_b, bn_e_g, bn_e_b):
    src, dst = edge_index[0], edge_index[1]
    for i in range(L):
        h_in, e_in = h, e
        A1h = _mm(h, A1_w[i], A1_b[i], 1000)
        A2h = _mm(h, A2_w[i], A2_b[i], 1000)
        A3h = _mm(h, A3_w[i], A3_b[i], 1000)
        B2h = _mm(h, B2_w[i], B2_b[i], 1000)
        B3h = _mm(h, B3_w[i], B3_b[i], 1000)
        B1e = _mm(e, B1_w[i], B1_b[i], 1000)
        e_ji = B1e + B2h[src] + B3h[dst]
        sigma_f = jax.nn.sigmoid(e_ji)
        zeros = jnp.zeros(h.shape, dtype=h.dtype)
        sum_sigma_h_f = zeros.at[dst].add(sigma_f * A2h[src])
        sum_sigma_f = zeros.at[dst].add(sigma_f)
        h_fwd = sum_sigma_h_f / (sum_sigma_f + 1e-6)
        e_ij = B1e + B2h[dst] + B3h[src]
        sigma_b = jax.nn.sigmoid(e_ij)
        sum_sigma_h_b = zeros.at[src].add(sigma_b * A3h[dst])
        sum_sigma_b = zeros.at[src].add(sigma_b)
        h_bwd = sum_sigma_h_b / (sum_sigma_b + 1e-6)
        h_new = A1h + h_fwd + h_bwd
        h_new = jax.nn.relu(_bn(h_new, bn_h_g[i], bn_h_b[i]))
        e_new = jax.nn.relu(_bn(e_ji, bn_e_g[i], bn_e_b[i]))
        h = h_in + h_new
        e = e_in + e_new
    return (h, e)
